# Initial kernel scaffold; baseline (speedup 1.0000x reference)
import jax
import jax.numpy as jnp
from jax.experimental import pallas as pl


def kernel(x, W_enc, b_enc, W_dec, b_dec):
    raise NotImplementedError("write your pallas kernel here")


# speedup vs baseline: 15.3887x; 15.3887x over previous
import jax
import jax.numpy as jnp
from jax.experimental import pallas as pl

_K = 64
_BLK = 256


def _body(x_ref, we_ref, be_ref, wd_ref, bd_ref, sparse_ref, recon_ref):
    lat = jnp.dot(x_ref[...], we_ref[...],
                  preferred_element_type=jnp.float32)
    lat = lat + be_ref[...]

    ik = jax.lax.bitcast_convert_type(lat, jnp.int32)
    keys = ik ^ ((ik >> 31) & jnp.int32(0x7FFFFFFF))

    cnt0 = jnp.sum((keys >= 0).astype(jnp.int32), axis=1, keepdims=True)
    base = jnp.where(cnt0 >= _K, jnp.int32(0), jnp.int32(-(2**31)))
    for b in range(30, -1, -1):
        trial = base | jnp.int32(1 << b)
        cnt = jnp.sum((keys >= trial).astype(jnp.int32), axis=1, keepdims=True)
        base = jnp.where(cnt >= _K, trial, base)

    sparse = jnp.where(keys >= base, lat, jnp.float32(0.0))
    sparse_ref[...] = sparse
    recon_ref[...] = jnp.dot(sparse.astype(jnp.bfloat16), wd_ref[...],
                             preferred_element_type=jnp.float32) + bd_ref[...]


@jax.jit
def kernel(x, W_enc, b_enc, W_dec, b_dec):
    B, D_in = x.shape
    D_lat = W_enc.shape[0]
    grid = (B // _BLK,)
    sparse, recon = pl.pallas_call(
        _body,
        grid=grid,
        in_specs=[
            pl.BlockSpec((_BLK, D_in), lambda i: (i, 0)),
            pl.BlockSpec((D_in, D_lat), lambda i: (0, 0)),
            pl.BlockSpec((1, D_lat), lambda i: (0, 0)),
            pl.BlockSpec((D_lat, D_in), lambda i: (0, 0)),
            pl.BlockSpec((1, D_in), lambda i: (0, 0)),
        ],
        out_specs=[
            pl.BlockSpec((_BLK, D_lat), lambda i: (i, 0)),
            pl.BlockSpec((_BLK, D_in), lambda i: (i, 0)),
        ],
        out_shape=[
            jax.ShapeDtypeStruct((B, D_lat), jnp.float32),
            jax.ShapeDtypeStruct((B, D_in), jnp.float32),
        ],
    )(x.astype(jnp.bfloat16), W_enc.T.astype(jnp.bfloat16), b_enc[None, :],
      W_dec.T.astype(jnp.bfloat16), b_dec[None, :])
    return (recon, sparse)
